# Initial kernel scaffold; baseline (speedup 1.0000x reference)
import jax
import jax.numpy as jnp
from jax.experimental import pallas as pl


def kernel(input, meshes, W1, b1, W2, b2):
    raise NotImplementedError("write your pallas kernel here")


# speedup vs baseline: 2.7533x; 2.7533x over previous
import functools

import jax
import jax.numpy as jnp
from jax import lax
from jax.experimental import pallas as pl
from jax.experimental.pallas import tpu as pltpu
from jax.experimental.pallas import tpu_sc as plsc

E = 160000
C = 256
NC = 2
NS = 16
NW = NC * NS
EPW = E // NW
CH = 40
NCHUNK = EPW // CH
LG = C // 16


def _sc_combine_body(xt_hbm, idx_hbm, out_hbm, i0_v, i1_v, i2_v, i3_v,
                     a_v, b_v, c_v, d_v, sem):
    wid = lax.axis_index("s") * NC + lax.axis_index("c")
    wbase = wid * EPW

    def chunk(ch, _):
        base = wbase + ch * CH
        pltpu.sync_copy(idx_hbm.at[pl.ds(0 * E + base, CH)], i0_v)
        pltpu.sync_copy(idx_hbm.at[pl.ds(1 * E + base, CH)], i1_v)
        pltpu.sync_copy(idx_hbm.at[pl.ds(2 * E + base, CH)], i2_v)
        pltpu.sync_copy(idx_hbm.at[pl.ds(3 * E + base, CH)], i3_v)
        cp0 = pltpu.async_copy(xt_hbm.at[i0_v], a_v, sem)
        cp1 = pltpu.async_copy(xt_hbm.at[i1_v], c_v, sem)
        cp2 = pltpu.async_copy(xt_hbm.at[i2_v], b_v, sem)
        cp3 = pltpu.async_copy(xt_hbm.at[i3_v], d_v, sem)
        cp0.wait()
        cp1.wait()
        cp2.wait()
        cp3.wait()

        def row(r, _):
            for g in range(LG):
                sl = pl.ds(g * 16, 16)
                a = a_v[r, sl]
                b = b_v[r, sl]
                a_v[r, sl] = a + b
                b_v[r, sl] = jnp.abs(a - b)
                c = c_v[r, sl]
                d = d_v[r, sl]
                c_v[r, sl] = c + d
                d_v[r, sl] = jnp.abs(c - d)
            return ()

        lax.fori_loop(0, CH, row, (), unroll=False)

        pltpu.sync_copy(a_v, out_hbm.at[0, pl.ds(base, CH)])
        pltpu.sync_copy(c_v, out_hbm.at[1, pl.ds(base, CH)])
        pltpu.sync_copy(b_v, out_hbm.at[2, pl.ds(base, CH)])
        pltpu.sync_copy(d_v, out_hbm.at[3, pl.ds(base, CH)])
        return ()

    lax.fori_loop(0, NCHUNK, chunk, (), unroll=False)


@jax.jit
def _sc_combine(xt, idx4):
    mesh = plsc.VectorSubcoreMesh(
        core_axis_name="c", subcore_axis_name="s", num_cores=NC, num_subcores=NS
    )
    f = pl.kernel(
        _sc_combine_body,
        out_type=jax.ShapeDtypeStruct((4, E, C), jnp.float32),
        mesh=mesh,
        scratch_types=[
            pltpu.VMEM((CH,), jnp.int32),
            pltpu.VMEM((CH,), jnp.int32),
            pltpu.VMEM((CH,), jnp.int32),
            pltpu.VMEM((CH,), jnp.int32),
            pltpu.VMEM((CH, C), jnp.float32),
            pltpu.VMEM((CH, C), jnp.float32),
            pltpu.VMEM((CH, C), jnp.float32),
            pltpu.VMEM((CH, C), jnp.float32),
            pltpu.SemaphoreType.DMA,
        ],
    )
    return f(xt, idx4)


def _mm1_body(xt_ref, cb_ref, w_ref, b_ref, o_ref):
    acc = jnp.dot(xt_ref[...], w_ref[0], preferred_element_type=jnp.float32)
    for j in range(4):
        acc += jnp.dot(cb_ref[j], w_ref[j + 1], preferred_element_type=jnp.float32)
    o_ref[...] = acc + b_ref[...]


@jax.jit
def _tc_mm1(xt, combo, w5, b):
    eb = 1280
    grid = (E // eb,)
    return pl.pallas_call(
        _mm1_body,
        grid=grid,
        in_specs=[
            pl.BlockSpec((eb, C), lambda i: (i, 0)),
            pl.BlockSpec((4, eb, C), lambda i: (0, i, 0)),
            pl.BlockSpec((5, C, C), lambda i: (0, 0, 0)),
            pl.BlockSpec((1, C), lambda i: (0, 0)),
        ],
        out_specs=pl.BlockSpec((eb, C), lambda i: (i, 0)),
        out_shape=jax.ShapeDtypeStruct((E, C), jnp.float32),
        compiler_params=pltpu.CompilerParams(
            dimension_semantics=("arbitrary",),
        ),
    )(xt, combo, w5, b)


def _mm2_body(h_ref, cb_ref, w_ref, b_ref, o_ref):
    dn = (((1,), (1,)), ((), ()))
    acc = lax.dot_general(w_ref[0], h_ref[...], dn, preferred_element_type=jnp.float32)
    for j in range(4):
        acc += lax.dot_general(w_ref[j + 1], cb_ref[j], dn,
                               preferred_element_type=jnp.float32)
    o_ref[...] = acc + b_ref[...]


@jax.jit
def _tc_mm2(h, combo, w5, b):
    eb = 1280
    grid = (E // eb,)
    return pl.pallas_call(
        _mm2_body,
        grid=grid,
        in_specs=[
            pl.BlockSpec((eb, C), lambda i: (i, 0)),
            pl.BlockSpec((4, eb, C), lambda i: (0, i, 0)),
            pl.BlockSpec((5, C, C), lambda i: (0, 0, 0)),
            pl.BlockSpec((C, 1), lambda i: (0, 0)),
        ],
        out_specs=pl.BlockSpec((C, eb), lambda i: (0, i)),
        out_shape=jax.ShapeDtypeStruct((C, E), jnp.float32),
        compiler_params=pltpu.CompilerParams(
            dimension_semantics=("arbitrary",),
        ),
    )(h, combo, w5, b)


def kernel(input, meshes, W1, b1, W2, b2):
    x = input[0]
    xt = x.T
    idx4 = meshes[0].astype(jnp.int32).T.reshape(4 * E)

    w1t = jnp.transpose(W1, (2, 1, 0))
    w2t = jnp.transpose(W2, (2, 0, 1))

    combo1 = _sc_combine(xt, idx4)
    h1 = _tc_mm1(xt, combo1, w1t, b1[None, :])
    combo2 = _sc_combine(h1, idx4)
    out = _tc_mm2(h1, combo2, w2t, b2[:, None])
    return out[None]
